# baseline (speedup 1.0000x reference)
import jax
import jax.numpy as jnp
from jax import lax
from jax.experimental import pallas as pl
from jax.experimental.pallas import tpu as pltpu
from jax.experimental.pallas import tpu_sc as plsc

_N = 32768
_E = 64
_K = 8
_L = 16
_NW = 32
_PER_W = _N // _NW
_CH = 256
_NCHUNK = _PER_W // _CH
_GROUPS = _CH // _L


def _tournament(leaves):
    while len(leaves) > 1:
        nxt = []
        for p in range(0, len(leaves), 2):
            (va, ia), (vb, ib) = leaves[p], leaves[p + 1]
            c = va >= vb
            nxt.append((jnp.where(c, va, vb), jnp.where(c, ia, ib)))
        leaves = nxt
    return leaves[0]


def _sc_body(x_hbm, bias_hbm, idx_hbm, w_hbm, xbuf, vt, idxbuf, wbuf, biasv):
    nc = plsc.get_sparse_core_info().num_cores
    wid = lax.axis_index("s") * nc + lax.axis_index("c")
    lane = jnp.arange(_L, dtype=jnp.int32)
    neg = jnp.full((_L,), -jnp.inf, dtype=jnp.float32)

    pltpu.sync_copy(bias_hbm, biasv)
    bvals = []
    for s in range(_E // _L):
        bvec = biasv[pl.ds(s * _L, _L)]
        bvals.extend(bvec[j] for j in range(_L))

    def chunk_body(c, carry):
        base = wid * _PER_W + c * _CH
        pltpu.sync_copy(x_hbm.at[pl.ds(base * _E, _CH * _E)], xbuf)

        def group_body(g, carry2):
            row = g * _L + lane
            rowe = row * _E
            for e in range(_E):
                vt[pl.ds(e * _L, _L)] = (
                    plsc.load_gather(xbuf, [rowe + e]) + bvals[e])
            vals, idxs = [], []
            for r in range(_K):
                subroots = []
                for s in range(4):
                    leaves = [(vt[pl.ds((s * 16 + j) * _L, _L)],
                               jnp.full((_L,), s * 16 + j, dtype=jnp.int32))
                              for j in range(16)]
                    subroots.append(_tournament(leaves))
                m, am = _tournament(subroots)
                vals.append(m)
                idxs.append(am)
                if r < _K - 1:
                    plsc.store_scatter(vt, [am * _L + lane], neg)
            es = [jnp.exp(v - vals[0]) for v in vals]
            ssum = es[0]
            for t in es[1:]:
                ssum = ssum + t
            rinv = 1.0 / ssum
            rowk = row * _K
            for r in range(_K):
                plsc.store_scatter(idxbuf, [rowk + r], idxs[r])
                plsc.store_scatter(wbuf, [rowk + r], es[r] * rinv)
            return carry2

        lax.fori_loop(0, _GROUPS, group_body, 0)
        pltpu.sync_copy(idxbuf, idx_hbm.at[pl.ds(base * _K, _CH * _K)])
        pltpu.sync_copy(wbuf, w_hbm.at[pl.ds(base * _K, _CH * _K)])
        return carry

    lax.fori_loop(0, _NCHUNK, chunk_body, 0)


@jax.jit
def kernel(gate_logits, bias):
    mesh = plsc.VectorSubcoreMesh(core_axis_name="c", subcore_axis_name="s")
    run = pl.kernel(
        _sc_body,
        out_type=[
            jax.ShapeDtypeStruct((_N * _K,), jnp.int32),
            jax.ShapeDtypeStruct((_N * _K,), jnp.float32),
        ],
        mesh=mesh,
        compiler_params=pltpu.CompilerParams(needs_layout_passes=False),
        scratch_types=[
            pltpu.VMEM((_CH * _E,), jnp.float32),
            pltpu.VMEM((_E * _L,), jnp.float32),
            pltpu.VMEM((_CH * _K,), jnp.int32),
            pltpu.VMEM((_CH * _K,), jnp.float32),
            pltpu.VMEM((_E,), jnp.float32),
        ],
    )
    idx, w = run(gate_logits.reshape(_N * _E), bias)
    return idx.reshape(_N, _K), w.reshape(_N, _K)


# speedup vs baseline: 1.3936x; 1.3936x over previous
import jax
import jax.numpy as jnp
from jax import lax
from jax.experimental import pallas as pl
from jax.experimental.pallas import tpu as pltpu
from jax.experimental.pallas import tpu_sc as plsc

_N = 32768
_E = 64
_K = 8
_L = 16
_NW = 32
_PER_W = _N // _NW
_CH = 256
_NCHUNK = _PER_W // _CH
_GROUPS = _CH // _L


def _tournament(leaves):
    while len(leaves) > 1:
        nxt = []
        for p in range(0, len(leaves), 2):
            (va, ia), (vb, ib) = leaves[p], leaves[p + 1]
            c = va >= vb
            nxt.append((jnp.where(c, va, vb), jnp.where(c, ia, ib)))
        leaves = nxt
    return leaves[0]


def _sc_body(x_hbm, idx_hbm, w_hbm, xbuf, idxbuf, wbuf):
    nc = plsc.get_sparse_core_info().num_cores
    wid = lax.axis_index("s") * nc + lax.axis_index("c")
    lane = jnp.arange(_L, dtype=jnp.int32)
    neg = jnp.full((_L,), -jnp.inf, dtype=jnp.float32)

    def chunk_body(c, carry):
        blk = wid * _NCHUNK + c
        pltpu.sync_copy(x_hbm.at[pl.ds(blk * _E * _CH, _E * _CH)], xbuf)

        def group_body(g, carry2):
            row = g * _L + lane
            g16 = g * _L
            vals, idxs = [], []
            for r in range(_K):
                subroots = []
                for s in range(4):
                    leaves = [(xbuf[pl.ds((s * 16 + j) * _CH + g16, _L)],
                               jnp.full((_L,), s * 16 + j, dtype=jnp.int32))
                              for j in range(16)]
                    subroots.append(_tournament(leaves))
                m, am = _tournament(subroots)
                vals.append(m)
                idxs.append(am)
                if r < _K - 1:
                    plsc.store_scatter(xbuf, [am * _CH + g16 + lane], neg)
            es = [jnp.exp(v - vals[0]) for v in vals]
            ssum = es[0]
            for t in es[1:]:
                ssum = ssum + t
            rinv = 1.0 / ssum
            rowk = row * _K
            for r in range(_K):
                plsc.store_scatter(idxbuf, [rowk + r], idxs[r])
                plsc.store_scatter(wbuf, [rowk + r], es[r] * rinv)
            return carry2

        lax.fori_loop(0, _GROUPS, group_body, 0)
        base = wid * _PER_W + c * _CH
        pltpu.sync_copy(idxbuf, idx_hbm.at[pl.ds(base * _K, _CH * _K)])
        pltpu.sync_copy(wbuf, w_hbm.at[pl.ds(base * _K, _CH * _K)])
        return carry

    lax.fori_loop(0, _NCHUNK, chunk_body, 0)


@jax.jit
def kernel(gate_logits, bias):
    xb = ((gate_logits + bias[None, :])
          .reshape(_NW * _NCHUNK, _CH, _E)
          .transpose(0, 2, 1)
          .reshape(_NW * _NCHUNK * _E * _CH))
    mesh = plsc.VectorSubcoreMesh(core_axis_name="c", subcore_axis_name="s")
    run = pl.kernel(
        _sc_body,
        out_type=[
            jax.ShapeDtypeStruct((_N * _K,), jnp.int32),
            jax.ShapeDtypeStruct((_N * _K,), jnp.float32),
        ],
        mesh=mesh,
        compiler_params=pltpu.CompilerParams(needs_layout_passes=False),
        scratch_types=[
            pltpu.VMEM((_E * _CH,), jnp.float32),
            pltpu.VMEM((_CH * _K,), jnp.int32),
            pltpu.VMEM((_CH * _K,), jnp.float32),
        ],
    )
    idx, w = run(xb)
    return idx.reshape(_N, _K), w.reshape(_N, _K)


# speedup vs baseline: 1.4786x; 1.0610x over previous
import jax
import jax.numpy as jnp
from jax import lax
from jax.experimental import pallas as pl
from jax.experimental.pallas import tpu as pltpu
from jax.experimental.pallas import tpu_sc as plsc

_N = 32768
_E = 64
_K = 8
_L = 16
_NW = 32
_PER_W = _N // _NW
_CH = 256
_NCHUNK = _PER_W // _CH
_GROUPS = _CH // _L


def _tournament(leaves):
    while len(leaves) > 1:
        nxt = []
        for p in range(0, len(leaves), 2):
            (va, ia), (vb, ib) = leaves[p], leaves[p + 1]
            c = va >= vb
            nxt.append((jnp.where(c, va, vb), jnp.where(c, ia, ib)))
        leaves = nxt
    return leaves[0]


def _sc_body(x_hbm, idx_hbm, w_hbm, xbuf, idxbuf, wbuf):
    nc = plsc.get_sparse_core_info().num_cores
    wid = lax.axis_index("s") * nc + lax.axis_index("c")
    lane = jnp.arange(_L, dtype=jnp.int32)
    neg = jnp.full((_L,), -jnp.inf, dtype=jnp.float32)

    def chunk_body(c, carry):
        blk = wid * _NCHUNK + c
        pltpu.sync_copy(x_hbm.at[pl.ds(blk * _E * _CH, _E * _CH)], xbuf)

        def group_body(g, carry2):
            row = g * _L + lane
            g16 = g * _L
            vals, idxs = [], []
            for r in range(_K):
                subroots = []
                for s in range(4):
                    leaves = [(xbuf[pl.ds((s * 16 + j) * _CH + g16, _L)],
                               jnp.full((_L,), s * 16 + j, dtype=jnp.int32))
                              for j in range(16)]
                    subroots.append(_tournament(leaves))
                m, am = _tournament(subroots)
                vals.append(m)
                idxs.append(am)
                if r < _K - 1:
                    plsc.store_scatter(xbuf, [am * _CH + g16 + lane], neg)
            es = [jnp.exp(v - vals[0]) for v in vals]
            ssum = es[0]
            for t in es[1:]:
                ssum = ssum + t
            rinv = 1.0 / ssum
            for r in range(_K):
                colr = jnp.full((_L,), r, dtype=jnp.int32)
                plsc.store_scatter(idxbuf, [row, colr], idxs[r])
                plsc.store_scatter(wbuf, [row, colr], es[r] * rinv)
            return carry2

        lax.fori_loop(0, _GROUPS, group_body, 0)
        base = wid * _PER_W + c * _CH
        pltpu.sync_copy(idxbuf, idx_hbm.at[pl.ds(base, _CH)])
        pltpu.sync_copy(wbuf, w_hbm.at[pl.ds(base, _CH)])
        return carry

    lax.fori_loop(0, _NCHUNK, chunk_body, 0)


@jax.jit
def kernel(gate_logits, bias):
    xb = ((gate_logits + bias[None, :])
          .reshape(_NW * _NCHUNK, _CH, _E)
          .transpose(0, 2, 1)
          .reshape(_NW * _NCHUNK * _E * _CH))
    mesh = plsc.VectorSubcoreMesh(core_axis_name="c", subcore_axis_name="s")
    run = pl.kernel(
        _sc_body,
        out_type=[
            jax.ShapeDtypeStruct((_N, _K), jnp.int32),
            jax.ShapeDtypeStruct((_N, _K), jnp.float32),
        ],
        mesh=mesh,
        compiler_params=pltpu.CompilerParams(needs_layout_passes=False),
        scratch_types=[
            pltpu.VMEM((_E * _CH,), jnp.float32),
            pltpu.VMEM((_CH, _K), jnp.int32),
            pltpu.VMEM((_CH, _K), jnp.float32),
        ],
    )
    idx, w = run(xb)
    return idx, w
